# Initial kernel scaffold; baseline (speedup 1.0000x reference)
import jax
import jax.numpy as jnp
from jax.experimental import pallas as pl


def kernel(x_paper, x_author, ei_cites, ei_writes, W_self, W_cites, W_writes, b_enc, W_head, b_head):
    raise NotImplementedError("write your pallas kernel here")


# speedup vs baseline: 6.4197x; 6.4197x over previous
import functools

import jax
import jax.numpy as jnp
from jax import lax
from jax.experimental import pallas as pl
from jax.experimental.pallas import tpu as pltpu
from jax.experimental.pallas import tpu_sc as plsc

N_PAPER = 10000
N_AUTHOR = 10000
E = 320000
D = 128
H = 256
C = 349

NC = 2
NS = 16
AW = 144
CHUNK = 125
GCH = 8
NGRP = E // (NS * CHUNK * GCH)
ROWS_PER_TILE = 640
ACC_ROWS = NS * ROWS_PER_TILE


def _sc_aggregate(xaug, srcs, dsts, zeros):
    mesh = plsc.VectorSubcoreMesh(core_axis_name="c", subcore_axis_name="s")

    @functools.partial(
        pl.kernel,
        mesh=mesh,
        out_type=jax.ShapeDtypeStruct((NC * NS, ROWS_PER_TILE, AW), jnp.float32),
        scratch_types=[
            pltpu.VMEM((GCH, CHUNK), jnp.int32),
            pltpu.VMEM((GCH, CHUNK), jnp.int32),
            pltpu.VMEM((CHUNK, AW), jnp.float32),
            pltpu.VMEM_SHARED((ACC_ROWS, AW), jnp.float32),
            pltpu.SemaphoreType.DMA,
        ],
        compiler_params=pltpu.CompilerParams(use_tc_tiling_on_sc=False),
    )
    def k(xaug_hbm, srcs_hbm, dsts_hbm, zeros_hbm, out_hbm,
          src_v, dst_v, rows_v, acc, sem):
        cid = lax.axis_index("c")
        sid = lax.axis_index("s")
        wid = cid * NS + sid
        pltpu.sync_copy(zeros_hbm, acc.at[pl.ds(sid * ROWS_PER_TILE, ROWS_PER_TILE)])
        plsc.subcore_barrier()

        def grp_body(g, carry):
            pltpu.sync_copy(srcs_hbm.at[wid * NGRP + g], src_v)
            pltpu.sync_copy(dsts_hbm.at[wid * NGRP + g], dst_v)

            def body(cix, carry2):
                pltpu.async_copy(xaug_hbm.at[src_v.at[cix]], rows_v, sem).wait()
                pltpu.sync_copy(rows_v, acc.at[dst_v.at[cix]], add=True)
                return carry2

            return lax.fori_loop(0, GCH, body, carry, unroll=False)

        lax.fori_loop(0, NGRP, grp_body, 0, unroll=False)
        plsc.subcore_barrier()
        pltpu.sync_copy(acc.at[pl.ds(sid * ROWS_PER_TILE, ROWS_PER_TILE)],
                        out_hbm.at[wid])

    return k(xaug, srcs, dsts, zeros)


def _dense_body(x_ref, pc_ref, pw_ref, w1_ref, b1_ref, w2_ref, b2_ref, o_ref):
    pc = pc_ref[...]
    pw = pw_ref[...]
    agg_c = pc[:, :D] / jnp.maximum(pc[:, D:D + 1], 1.0)
    agg_w = pw[:, :D] / jnp.maximum(pw[:, D:D + 1], 1.0)
    xin = jnp.concatenate([x_ref[...], agg_c, agg_w], axis=1)
    h = jnp.dot(xin, w1_ref[...], preferred_element_type=jnp.float32) + b1_ref[...]
    h = jnp.maximum(h, 0.0)
    o_ref[...] = jnp.dot(h, w2_ref[...], preferred_element_type=jnp.float32) + b2_ref[...]


def kernel(x_paper, x_author, ei_cites, ei_writes, W_self, W_cites, W_writes,
           b_enc, W_head, b_head):
    ones = jnp.ones((N_PAPER, 1), jnp.float32)
    pad = jnp.zeros((N_PAPER, AW - D - 1), jnp.float32)
    xaug = jnp.concatenate([
        jnp.concatenate([x_paper, ones, pad], axis=1),
        jnp.concatenate([x_author, ones, pad], axis=1),
    ], axis=0)
    srcs = jnp.concatenate([ei_cites[0], ei_writes[0] + N_PAPER]
                           ).reshape(NC * NS * NGRP, GCH, CHUNK)
    dsts = jnp.concatenate([ei_cites[1], ei_writes[1]]
                           ).reshape(NC * NS * NGRP, GCH, CHUNK)
    zeros = jnp.zeros((ROWS_PER_TILE, AW), jnp.float32)

    parts = _sc_aggregate(xaug, srcs, dsts, zeros)
    part_c = parts[:NS].reshape(ACC_ROWS, AW)[:N_PAPER]
    part_w = parts[NS:].reshape(ACC_ROWS, AW)[:N_PAPER]

    W1 = jnp.concatenate([W_self, W_cites, W_writes], axis=0)
    b1 = b_enc.reshape(1, H)
    b2 = b_head.reshape(1, C)

    BR = 1000
    out = pl.pallas_call(
        _dense_body,
        grid=(N_PAPER // BR,),
        in_specs=[
            pl.BlockSpec((BR, D), lambda i: (i, 0)),
            pl.BlockSpec((BR, AW), lambda i: (i, 0)),
            pl.BlockSpec((BR, AW), lambda i: (i, 0)),
            pl.BlockSpec((3 * D, H), lambda i: (0, 0)),
            pl.BlockSpec((1, H), lambda i: (0, 0)),
            pl.BlockSpec((H, C), lambda i: (0, 0)),
            pl.BlockSpec((1, C), lambda i: (0, 0)),
        ],
        out_specs=pl.BlockSpec((BR, C), lambda i: (i, 0)),
        out_shape=jax.ShapeDtypeStruct((N_PAPER, C), jnp.float32),
    )(x_paper, part_c, part_w, W1, b1, W_head, b2)
    return out
